# baseline (speedup 1.0000x reference)
import functools

import jax
import jax.numpy as jnp
from jax import lax
from jax.experimental import pallas as pl
from jax.experimental.pallas import tpu as pltpu
from jax.experimental.pallas import tpu_sc as plsc

NUM_EMBEDDINGS = 1000000
EMB_DIM = 64
NUM_EDGE_TYPES = 8
LANES = 16
FEAT_CHUNKS = EMB_DIM // LANES
PACK = 2
CHUNK_ROWS = 256
STREAM_IDX = 128


def _shuffle(x, idx):
    dnums = lax.GatherDimensionNumbers(
        offset_dims=(), collapsed_slice_dims=(0,), start_index_map=(0,))
    return lax.gather(
        x, idx[:, None], dnums, slice_sizes=(1,),
        mode=lax.GatherScatterMode.PROMISE_IN_BOUNDS)


@functools.lru_cache(maxsize=None)
def _build(batch: int):
    info = plsc.get_sparse_core_info()
    nc, ns = info.num_cores, info.num_subcores
    nw = nc * ns
    e_per_w = batch // nw
    rows_per_w = 2 * e_per_w
    n_chunks = rows_per_w // CHUNK_ROWS
    chunk_edges = CHUNK_ROWS // 2
    packed_dim = PACK * EMB_DIM

    mesh = plsc.VectorSubcoreMesh(core_axis_name="c", subcore_axis_name="s")

    @functools.partial(
        pl.kernel,
        mesh=mesh,
        out_type=jax.ShapeDtypeStruct((batch,), jnp.float32),
        scratch_types=[
            pltpu.VMEM((rows_per_w,), jnp.int32),
            pltpu.VMEM((rows_per_w,), jnp.int32),
            pltpu.VMEM((CHUNK_ROWS, packed_dim), jnp.float32),
            pltpu.VMEM((CHUNK_ROWS, packed_dim), jnp.float32),
            pltpu.VMEM((e_per_w,), jnp.int32),
            pltpu.VMEM((NUM_EDGE_TYPES, EMB_DIM), jnp.float32),
            pltpu.VMEM((NUM_EDGE_TYPES, EMB_DIM), jnp.float32),
            pltpu.VMEM((e_per_w,), jnp.float32),
            pltpu.SemaphoreType.DMA,
            pltpu.SemaphoreType.DMA,
        ],
    )
    def sc_kernel(pairs_hbm, et_hbm, packed_hbm, t_hbm, d_hbm, out_hbm,
                  idx_v, pid_v, rows_a, rows_b, et_v, t_v, d_v, scores_v,
                  sem_a, sem_b):
        wid = lax.axis_index("s") * nc + lax.axis_index("c")
        ebase = wid * e_per_w

        pltpu.sync_copy(pairs_hbm.at[pl.ds(ebase * 2, rows_per_w)], idx_v)
        pltpu.sync_copy(et_hbm.at[pl.ds(ebase, e_per_w)], et_v)
        pltpu.sync_copy(t_hbm, t_v)
        pltpu.sync_copy(d_hbm, d_v)

        def pid_body(b, carry):
            ids = idx_v[pl.ds(b * LANES, LANES)]
            pid_v[pl.ds(b * LANES, LANES)] = lax.shift_right_logical(ids, 1)
            return carry
        lax.fori_loop(0, rows_per_w // LANES, pid_body, 0)

        def fire_chunk(chunk, rows_ref, sem):
            copies = []
            for g in range(CHUNK_ROWS // STREAM_IDX):
                o = chunk * CHUNK_ROWS + g * STREAM_IDX
                copies.append(pltpu.async_copy(
                    packed_hbm.at[pid_v.at[pl.ds(o, STREAM_IDX)]],
                    rows_ref.at[pl.ds(g * STREAM_IDX, STREAM_IDX), :],
                    sem))
            return copies

        lane_iota = lax.iota(jnp.int32, LANES)
        perms = [lane_iota ^ s for s in (8, 4, 2, 1)]

        def compute_chunk(chunk, rows_ref):
            def block_body(b, carry):
                e0 = chunk * chunk_edges + b * LANES
                et_vec = et_v[pl.ds(e0, LANES)]
                ids_s = idx_v[pl.ds(2 * e0, LANES)]
                ids_s2 = idx_v[pl.ds(2 * e0 + LANES, LANES)]
                score_vec = jnp.zeros((LANES,), jnp.float32)
                for j in range(LANES):
                    le = b * LANES + j
                    et = et_vec[j]
                    idsrc = ids_s[2 * j] if j < 8 else ids_s2[2 * j - LANES]
                    iddst = ids_s[2 * j + 1] if j < 8 else ids_s2[2 * j + 1 - LANES]
                    soff = (idsrc & 1) * EMB_DIM
                    doff = (iddst & 1) * EMB_DIM
                    acc = jnp.zeros((LANES,), jnp.float32)
                    for c in range(FEAT_CHUNKS):
                        fsl = pl.ds(c * LANES, LANES)
                        src = rows_ref[2 * le, pl.ds(soff + c * LANES, LANES)]
                        dst = rows_ref[2 * le + 1, pl.ds(doff + c * LANES, LANES)]
                        tt = t_v[et, fsl]
                        dd = d_v[et, fsl]
                        acc = acc + (src + tt) * (dst * dd)
                    for p in perms:
                        acc = acc + _shuffle(acc, p)
                    score_vec = jnp.where(lane_iota == j, acc, score_vec)
                scores_v[pl.ds(e0, LANES)] = score_vec
                return carry
            lax.fori_loop(0, chunk_edges // LANES, block_body, 0)

        bufs = (rows_a, rows_b)
        sems = (sem_a, sem_b)
        inflight = {0: fire_chunk(0, bufs[0], sems[0])}
        for chunk in range(n_chunks):
            if chunk + 1 < n_chunks:
                inflight[chunk + 1] = fire_chunk(
                    chunk + 1, bufs[(chunk + 1) % 2], sems[(chunk + 1) % 2])
            for cp in inflight.pop(chunk):
                cp.wait()
            compute_chunk(chunk, bufs[chunk % 2])

        pltpu.sync_copy(scores_v, out_hbm.at[pl.ds(ebase, e_per_w)])

    return sc_kernel


def kernel(src_dst_pairs, condensed_edge_types, table, src_translation, dst_diag):
    batch = condensed_edge_types.shape[0]
    n_emb, emb_dim = table.shape
    packed = table.reshape(n_emb // PACK, PACK * emb_dim)
    fn = _build(batch)
    return fn(
        jnp.asarray(src_dst_pairs, jnp.int32),
        jnp.asarray(condensed_edge_types, jnp.int32),
        packed,
        src_translation,
        dst_diag,
    )


# speedup vs baseline: 1.9008x; 1.9008x over previous
import functools

import jax
import jax.numpy as jnp
from jax import lax
from jax.experimental import pallas as pl
from jax.experimental.pallas import tpu as pltpu
from jax.experimental.pallas import tpu_sc as plsc

NUM_EMBEDDINGS = 1000000
EMB_DIM = 64
NUM_EDGE_TYPES = 8
LANES = 16

NTILES = (NUM_EMBEDDINGS + 127) // 128
JBT = 32
NJB = (NTILES + JBT - 1) // JBT
LAST_T = NTILES - (NJB - 1) * JBT
NGRP = EMB_DIM // 8
NWIN = NGRP * NJB
RAW_ROWS = NGRP * NTILES * 8
GSTRIDE = NTILES * 8 * 128

CEDGES = 128
EPC = CEDGES * EMB_DIM


def _shuffle(x, idx):
    dnums = lax.GatherDimensionNumbers(
        offset_dims=(), collapsed_slice_dims=(0,), start_index_map=(0,))
    return lax.gather(
        x, idx[:, None], dnums, slice_sizes=(1,),
        mode=lax.GatherScatterMode.PROMISE_IN_BOUNDS)


@functools.lru_cache(maxsize=None)
def _build_pack():
    info = plsc.get_sparse_core_info()
    nc, ns = info.num_cores, info.num_subcores
    nw = nc * ns
    slots = (NWIN + nw - 1) // nw

    mesh = plsc.VectorSubcoreMesh(core_axis_name="c", subcore_axis_name="s")

    @functools.partial(
        pl.kernel,
        mesh=mesh,
        out_type=jax.ShapeDtypeStruct((RAW_ROWS, 128), jnp.float32),
        scratch_types=[
            pltpu.VMEM((8, JBT * 128), jnp.float32),
            pltpu.SemaphoreType.DMA,
        ],
        compiler_params=pltpu.CompilerParams(disable_bounds_checks=True),
    )
    def pack_kernel(tt_hbm, raw_hbm, stage, sem):
        wid = lax.axis_index("s") * nc + lax.axis_index("c")

        def do_window(g, jb, ntiles):
            width = ntiles * 128
            pltpu.sync_copy(
                tt_hbm.at[pl.ds(g * 8, 8), pl.ds(jb * (JBT * 128), width)],
                stage.at[:, pl.ds(0, width)])
            q0 = g * (NTILES * 8) + jb * (JBT * 8)
            copies = []
            for w in range(ntiles):
                copies.append(pltpu.async_copy(
                    stage.at[:, pl.ds(w * 128, 128)],
                    raw_hbm.at[pl.ds(q0 + w * 8, 8), :],
                    sem))
            for cp in copies:
                cp.wait()

        def body(slot, carry):
            win = wid + nw * slot
            g = win // NJB
            jb = win - g * NJB

            @pl.when(jnp.logical_and(win < NWIN, jb < NJB - 1))
            def _():
                do_window(g, jb, JBT)

            @pl.when(jnp.logical_and(win < NWIN, jb == NJB - 1))
            def _():
                do_window(g, jb, LAST_T)
            return carry

        lax.fori_loop(0, slots, body, 0)

    return pack_kernel


@functools.lru_cache(maxsize=None)
def _build_score(batch: int):
    info = plsc.get_sparse_core_info()
    nc, ns = info.num_cores, info.num_subcores
    nw = nc * ns
    e_per_w = batch // nw
    n_chunks = e_per_w // CEDGES
    blocks_per_chunk = CEDGES // LANES

    mesh = plsc.VectorSubcoreMesh(core_axis_name="c", subcore_axis_name="s")

    @functools.partial(
        pl.kernel,
        mesh=mesh,
        out_type=jax.ShapeDtypeStruct((batch,), jnp.float32),
        scratch_types=[
            pltpu.VMEM((e_per_w,), jnp.int32),
            pltpu.VMEM((e_per_w,), jnp.int32),
            pltpu.VMEM((e_per_w,), jnp.int32),
            pltpu.VMEM((e_per_w * EMB_DIM,), jnp.int32),
            pltpu.VMEM((e_per_w * EMB_DIM,), jnp.int32),
            pltpu.VMEM((EPC,), jnp.float32),
            pltpu.VMEM((EPC,), jnp.float32),
            pltpu.VMEM((EPC,), jnp.float32),
            pltpu.VMEM((EPC,), jnp.float32),
            pltpu.VMEM((EMB_DIM, LANES), jnp.float32),
            pltpu.VMEM((EMB_DIM, LANES), jnp.float32),
            pltpu.VMEM((e_per_w,), jnp.float32),
            pltpu.SemaphoreType.DMA,
            pltpu.SemaphoreType.DMA,
        ],
    )
    def score_kernel(sid_hbm, did_hbm, et_hbm, flat_hbm, tp_hbm, dp_hbm,
                     out_hbm, sid_v, did_v, et_v, eis_v, eid_v,
                     sf_a, sf_b, df_a, df_b, tp_v, dp_v, scores_v,
                     sem_a, sem_b):
        wid = lax.axis_index("s") * nc + lax.axis_index("c")
        ebase = wid * e_per_w

        pltpu.sync_copy(sid_hbm.at[pl.ds(ebase, e_per_w)], sid_v)
        pltpu.sync_copy(did_hbm.at[pl.ds(ebase, e_per_w)], did_v)
        pltpu.sync_copy(et_hbm.at[pl.ds(ebase, e_per_w)], et_v)
        pltpu.sync_copy(tp_hbm, tp_v)
        pltpu.sync_copy(dp_hbm, dp_v)

        cconst = [(c // 8) * GSTRIDE + (c % 8) * 128 for c in range(EMB_DIM)]

        def build_body(b, carry):
            chunk = b // blocks_per_chunk
            off0 = chunk * EPC + (b - chunk * blocks_per_chunk) * LANES
            for ids, ei in ((sid_v, eis_v), (did_v, eid_v)):
                ivec = ids[pl.ds(b * LANES, LANES)]
                base = (lax.shift_right_logical(ivec, 7) * 1024
                        + (ivec & 127))
                for c in range(EMB_DIM):
                    ei[pl.ds(off0 + c * CEDGES, LANES)] = base + cconst[c]
            return carry
        lax.fori_loop(0, e_per_w // LANES, build_body, 0)

        def fire_chunk(chunk, ei, dstbuf, sem):
            def fb(k, carry):
                o = chunk * EPC + k * 128
                pltpu.async_copy(
                    flat_hbm.at[ei.at[pl.ds(o, 128)]],
                    dstbuf.at[pl.ds(k * 128, 128)],
                    sem)
                return carry
            lax.fori_loop(0, EPC // 128, fb, 0)

        def wait_chunk(chunk, ei, dstbuf, sem):
            def wb(k, carry):
                o = chunk * EPC + k * 128
                pltpu.make_async_copy(
                    flat_hbm.at[ei.at[pl.ds(o, 128)]],
                    dstbuf.at[pl.ds(k * 128, 128)],
                    sem).wait()
                return carry
            lax.fori_loop(0, EPC // 128, wb, 0)

        def compute_chunk(chunk, sbuf, dbuf):
            def cb(b, carry):
                e0 = chunk * CEDGES + b * LANES
                et_vec = et_v[pl.ds(e0, LANES)]
                acc = jnp.zeros((LANES,), jnp.float32)
                for c in range(EMB_DIM):
                    sl = pl.ds(c * CEDGES + b * LANES, LANES)
                    sv = sbuf[sl]
                    dv = dbuf[sl]
                    tt = _shuffle(tp_v[c], et_vec)
                    dd = _shuffle(dp_v[c], et_vec)
                    acc = acc + (sv + tt) * (dv * dd)
                scores_v[pl.ds(e0, LANES)] = acc
                return carry
            lax.fori_loop(0, blocks_per_chunk, cb, 0)

        sbufs = (sf_a, sf_b)
        dbufs = (df_a, df_b)
        fire_chunk(0, eis_v, sbufs[0], sem_a)
        fire_chunk(0, eid_v, dbufs[0], sem_b)
        for chunk in range(n_chunks):
            if chunk + 1 < n_chunks:
                fire_chunk(chunk + 1, eis_v, sbufs[(chunk + 1) % 2], sem_a)
                fire_chunk(chunk + 1, eid_v, dbufs[(chunk + 1) % 2], sem_b)
            wait_chunk(chunk, eis_v, sbufs[chunk % 2], sem_a)
            wait_chunk(chunk, eid_v, dbufs[chunk % 2], sem_b)
            compute_chunk(chunk, sbufs[chunk % 2], dbufs[chunk % 2])

        pltpu.sync_copy(scores_v, out_hbm.at[pl.ds(ebase, e_per_w)])

    return score_kernel


def kernel(src_dst_pairs, condensed_edge_types, table, src_translation, dst_diag):
    batch = condensed_edge_types.shape[0]
    pairs2 = jnp.asarray(src_dst_pairs, jnp.int32).reshape(batch, 2)
    src_ids = pairs2[:, 0]
    dst_ids = pairs2[:, 1]
    tpad = jnp.pad(src_translation.T, ((0, 0), (0, LANES - NUM_EDGE_TYPES)))
    dpad = jnp.pad(dst_diag.T, ((0, 0), (0, LANES - NUM_EDGE_TYPES)))
    raw = _build_pack()(table.T)
    flat = raw.reshape(-1)
    return _build_score(batch)(
        src_ids,
        dst_ids,
        jnp.asarray(condensed_edge_types, jnp.int32),
        flat,
        tpad,
        dpad,
    )


# speedup vs baseline: 1.9044x; 1.0019x over previous
import functools

import jax
import jax.numpy as jnp
from jax import lax
from jax.experimental import pallas as pl
from jax.experimental.pallas import tpu as pltpu
from jax.experimental.pallas import tpu_sc as plsc

NUM_EMBEDDINGS = 1000000
EMB_DIM = 64
NUM_EDGE_TYPES = 8
LANES = 16

NTILES = (NUM_EMBEDDINGS + 127) // 128
JBT = 32
NJB = (NTILES + JBT - 1) // JBT
LAST_T = NTILES - (NJB - 1) * JBT
NGRP = EMB_DIM // 8
NWIN = NGRP * NJB
RAW_ROWS = NGRP * NTILES * 8
GSTRIDE = NTILES * 8 * 128

CEDGES = 128
EPC = CEDGES * EMB_DIM


def _shuffle(x, idx):
    dnums = lax.GatherDimensionNumbers(
        offset_dims=(), collapsed_slice_dims=(0,), start_index_map=(0,))
    return lax.gather(
        x, idx[:, None], dnums, slice_sizes=(1,),
        mode=lax.GatherScatterMode.PROMISE_IN_BOUNDS)


@functools.lru_cache(maxsize=None)
def _build_pack():
    info = plsc.get_sparse_core_info()
    nc, ns = info.num_cores, info.num_subcores
    nw = nc * ns
    slots = (NWIN + nw - 1) // nw

    mesh = plsc.VectorSubcoreMesh(core_axis_name="c", subcore_axis_name="s")

    @functools.partial(
        pl.kernel,
        mesh=mesh,
        out_type=jax.ShapeDtypeStruct((RAW_ROWS, 128), jnp.float32),
        scratch_types=[
            pltpu.VMEM((8, JBT * 128), jnp.float32),
            pltpu.SemaphoreType.DMA,
        ],
        compiler_params=pltpu.CompilerParams(disable_bounds_checks=True),
    )
    def pack_kernel(tt_hbm, raw_hbm, stage, sem):
        wid = lax.axis_index("s") * nc + lax.axis_index("c")

        def do_window(g, jb, ntiles):
            width = ntiles * 128
            pltpu.sync_copy(
                tt_hbm.at[pl.ds(g * 8, 8), pl.ds(jb * (JBT * 128), width)],
                stage.at[:, pl.ds(0, width)])
            q0 = g * (NTILES * 8) + jb * (JBT * 8)
            copies = []
            for w in range(ntiles):
                copies.append(pltpu.async_copy(
                    stage.at[:, pl.ds(w * 128, 128)],
                    raw_hbm.at[pl.ds(q0 + w * 8, 8), :],
                    sem))
            for cp in copies:
                cp.wait()

        def body(slot, carry):
            win = wid + nw * slot
            g = win // NJB
            jb = win - g * NJB

            @pl.when(jnp.logical_and(win < NWIN, jb < NJB - 1))
            def _():
                do_window(g, jb, JBT)

            @pl.when(jnp.logical_and(win < NWIN, jb == NJB - 1))
            def _():
                do_window(g, jb, LAST_T)
            return carry

        lax.fori_loop(0, slots, body, 0)

    return pack_kernel


@functools.lru_cache(maxsize=None)
def _build_score(batch: int):
    info = plsc.get_sparse_core_info()
    nc, ns = info.num_cores, info.num_subcores
    nw = nc * ns
    e_per_w = batch // nw
    n_chunks = e_per_w // CEDGES
    blocks_per_chunk = CEDGES // LANES

    mesh = plsc.VectorSubcoreMesh(core_axis_name="c", subcore_axis_name="s")

    @functools.partial(
        pl.kernel,
        mesh=mesh,
        out_type=jax.ShapeDtypeStruct((batch,), jnp.float32),
        scratch_types=[
            pltpu.VMEM((e_per_w,), jnp.int32),
            pltpu.VMEM((e_per_w,), jnp.int32),
            pltpu.VMEM((e_per_w,), jnp.int32),
            pltpu.VMEM((e_per_w * EMB_DIM,), jnp.int32),
            pltpu.VMEM((e_per_w * EMB_DIM,), jnp.int32),
            pltpu.VMEM((EPC,), jnp.float32),
            pltpu.VMEM((EPC,), jnp.float32),
            pltpu.VMEM((EPC,), jnp.float32),
            pltpu.VMEM((EPC,), jnp.float32),
            pltpu.VMEM((EMB_DIM, LANES), jnp.float32),
            pltpu.VMEM((EMB_DIM, LANES), jnp.float32),
            pltpu.VMEM((e_per_w,), jnp.float32),
            pltpu.SemaphoreType.DMA,
            pltpu.SemaphoreType.DMA,
        ],
    )
    def score_kernel(sid_hbm, did_hbm, et_hbm, flat_hbm, tp_hbm, dp_hbm,
                     out_hbm, sid_v, did_v, et_v, eis_v, eid_v,
                     sf_a, sf_b, df_a, df_b, tp_v, dp_v, scores_v,
                     sem_a, sem_b):
        wid = lax.axis_index("s") * nc + lax.axis_index("c")
        ebase = wid * e_per_w

        pltpu.sync_copy(sid_hbm.at[pl.ds(ebase, e_per_w)], sid_v)
        pltpu.sync_copy(did_hbm.at[pl.ds(ebase, e_per_w)], did_v)
        pltpu.sync_copy(et_hbm.at[pl.ds(ebase, e_per_w)], et_v)
        pltpu.sync_copy(tp_hbm, tp_v)
        pltpu.sync_copy(dp_hbm, dp_v)

        cconst = [(c // 8) * GSTRIDE + (c % 8) * 128 for c in range(EMB_DIM)]

        def build_body(b, carry):
            chunk = b // blocks_per_chunk
            off0 = chunk * EPC + (b - chunk * blocks_per_chunk) * LANES
            for ids, ei in ((sid_v, eis_v), (did_v, eid_v)):
                ivec = ids[pl.ds(b * LANES, LANES)]
                base = (lax.shift_right_logical(ivec, 7) * 1024
                        + (ivec & 127))
                for c in range(EMB_DIM):
                    ei[pl.ds(off0 + c * CEDGES, LANES)] = base + cconst[c]
            return carry
        lax.fori_loop(0, e_per_w // LANES, build_body, 0)

        def fire_chunk(chunk, ei, dstbuf, sem):
            pltpu.async_copy(
                flat_hbm.at[ei.at[pl.ds(chunk * EPC, EPC)]],
                dstbuf,
                sem)

        def wait_chunk(chunk, ei, dstbuf, sem):
            pltpu.make_async_copy(
                flat_hbm.at[ei.at[pl.ds(chunk * EPC, EPC)]],
                dstbuf,
                sem).wait()

        def compute_chunk(chunk, sbuf, dbuf):
            def cb(b, carry):
                e0 = chunk * CEDGES + b * LANES
                et_vec = et_v[pl.ds(e0, LANES)]
                acc = jnp.zeros((LANES,), jnp.float32)
                for c in range(EMB_DIM):
                    sl = pl.ds(c * CEDGES + b * LANES, LANES)
                    sv = sbuf[sl]
                    dv = dbuf[sl]
                    tt = _shuffle(tp_v[c], et_vec)
                    dd = _shuffle(dp_v[c], et_vec)
                    acc = acc + (sv + tt) * (dv * dd)
                scores_v[pl.ds(e0, LANES)] = acc
                return carry
            lax.fori_loop(0, blocks_per_chunk, cb, 0)

        sbufs = (sf_a, sf_b)
        dbufs = (df_a, df_b)
        fire_chunk(0, eis_v, sbufs[0], sem_a)
        fire_chunk(0, eid_v, dbufs[0], sem_b)
        for chunk in range(n_chunks):
            if chunk + 1 < n_chunks:
                fire_chunk(chunk + 1, eis_v, sbufs[(chunk + 1) % 2], sem_a)
                fire_chunk(chunk + 1, eid_v, dbufs[(chunk + 1) % 2], sem_b)
            wait_chunk(chunk, eis_v, sbufs[chunk % 2], sem_a)
            wait_chunk(chunk, eid_v, dbufs[chunk % 2], sem_b)
            compute_chunk(chunk, sbufs[chunk % 2], dbufs[chunk % 2])

        pltpu.sync_copy(scores_v, out_hbm.at[pl.ds(ebase, e_per_w)])

    return score_kernel


def kernel(src_dst_pairs, condensed_edge_types, table, src_translation, dst_diag):
    batch = condensed_edge_types.shape[0]
    pairs2 = jnp.asarray(src_dst_pairs, jnp.int32).reshape(batch, 2)
    src_ids = pairs2[:, 0]
    dst_ids = pairs2[:, 1]
    tpad = jnp.pad(src_translation.T, ((0, 0), (0, LANES - NUM_EDGE_TYPES)))
    dpad = jnp.pad(dst_diag.T, ((0, 0), (0, LANES - NUM_EDGE_TYPES)))
    raw = _build_pack()(table.T)
    flat = raw.reshape(-1)
    return _build_score(batch)(
        src_ids,
        dst_ids,
        jnp.asarray(condensed_edge_types, jnp.int32),
        flat,
        tpad,
        dpad,
    )


# speedup vs baseline: 2.1016x; 1.1035x over previous
import functools

import jax
import jax.numpy as jnp
from jax import lax
from jax.experimental import pallas as pl
from jax.experimental.pallas import tpu as pltpu
from jax.experimental.pallas import tpu_sc as plsc

NUM_EMBEDDINGS = 1000000
EMB_DIM = 64
NUM_EDGE_TYPES = 8
LANES = 16

NTILES = (NUM_EMBEDDINGS + 127) // 128
JBT = 32
NJB = (NTILES + JBT - 1) // JBT
LAST_T = NTILES - (NJB - 1) * JBT
NGRP = EMB_DIM // 8
NWIN = NGRP * NJB
RAW_ROWS = NGRP * NTILES * 8
GSTRIDE = NTILES * 8 * 128

CEDGES = 128
EPC = CEDGES * EMB_DIM


def _shuffle(x, idx):
    dnums = lax.GatherDimensionNumbers(
        offset_dims=(), collapsed_slice_dims=(0,), start_index_map=(0,))
    return lax.gather(
        x, idx[:, None], dnums, slice_sizes=(1,),
        mode=lax.GatherScatterMode.PROMISE_IN_BOUNDS)


@functools.lru_cache(maxsize=None)
def _build_pack():
    info = plsc.get_sparse_core_info()
    nc, ns = info.num_cores, info.num_subcores
    nw = nc * ns
    slots = (NWIN + nw - 1) // nw

    mesh = plsc.VectorSubcoreMesh(core_axis_name="c", subcore_axis_name="s")

    @functools.partial(
        pl.kernel,
        mesh=mesh,
        out_type=jax.ShapeDtypeStruct((RAW_ROWS, 128), jnp.float32),
        scratch_types=[
            pltpu.VMEM((8, JBT * 128), jnp.float32),
            pltpu.VMEM((8, JBT * 128), jnp.float32),
            pltpu.SemaphoreType.DMA,
            pltpu.SemaphoreType.DMA,
            pltpu.SemaphoreType.DMA,
            pltpu.SemaphoreType.DMA,
        ],
        compiler_params=pltpu.CompilerParams(disable_bounds_checks=True),
    )
    def pack_kernel(tt_hbm, raw_hbm, st_a, st_b, sin_a, sin_b, sout_a, sout_b):
        wid = lax.axis_index("s") * nc + lax.axis_index("c")
        stages = (st_a, st_b)
        sins = (sin_a, sin_b)
        souts = (sout_a, sout_b)

        def in_desc(g, jb, ntiles, p):
            width = ntiles * 128
            return pltpu.make_async_copy(
                tt_hbm.at[pl.ds(g * 8, 8), pl.ds(jb * (JBT * 128), width)],
                stages[p].at[:, pl.ds(0, width)], sins[p])

        def out_desc(g, jb, w, p):
            q0 = g * (NTILES * 8) + jb * (JBT * 8)
            return pltpu.make_async_copy(
                stages[p].at[:, pl.ds(w * 128, 128)],
                raw_hbm.at[pl.ds(q0 + w * 8, 8), :], souts[p])

        def guarded(slot, fn):
            win = wid + nw * slot
            g = win // NJB
            jb = win - g * NJB
            valid = jnp.logical_and(win >= 0, win < NWIN)

            @pl.when(jnp.logical_and(valid, jb < NJB - 1))
            def _():
                fn(g, jb, JBT)

            @pl.when(jnp.logical_and(valid, jb == NJB - 1))
            def _():
                fn(g, jb, LAST_T)

        def fire_in(slot, p):
            guarded(slot, lambda g, jb, nt: in_desc(g, jb, nt, p).start())

        def wait_in(slot, p):
            guarded(slot, lambda g, jb, nt: in_desc(g, jb, nt, p).wait())

        def fire_outs(slot, p):
            def go(g, jb, nt):
                for w in range(nt):
                    out_desc(g, jb, w, p).start()
            guarded(slot, go)

        def wait_outs(slot, p):
            def go(g, jb, nt):
                for w in range(nt):
                    out_desc(g, jb, w, p).wait()
            guarded(slot, go)

        fire_in(0, 0)

        def pair_body(m, carry):
            sa = 2 * m
            sb = sa + 1
            wait_in(sa, 0)
            fire_outs(sa, 0)
            wait_outs(sb - 2, 1)
            fire_in(sb, 1)
            wait_in(sb, 1)
            fire_outs(sb, 1)
            wait_outs(sa, 0)
            fire_in(sa + 2, 0)
            return carry

        lax.fori_loop(0, slots // 2, pair_body, 0)
        wait_outs(slots - 1, 1)

    return pack_kernel


@functools.lru_cache(maxsize=None)
def _build_score(batch: int):
    info = plsc.get_sparse_core_info()
    nc, ns = info.num_cores, info.num_subcores
    nw = nc * ns
    e_per_w = batch // nw
    n_chunks = e_per_w // CEDGES
    blocks_per_chunk = CEDGES // LANES

    mesh = plsc.VectorSubcoreMesh(core_axis_name="c", subcore_axis_name="s")

    @functools.partial(
        pl.kernel,
        mesh=mesh,
        out_type=jax.ShapeDtypeStruct((batch,), jnp.float32),
        scratch_types=[
            pltpu.VMEM((e_per_w,), jnp.int32),
            pltpu.VMEM((e_per_w,), jnp.int32),
            pltpu.VMEM((e_per_w,), jnp.int32),
            pltpu.VMEM((e_per_w * EMB_DIM,), jnp.int32),
            pltpu.VMEM((e_per_w * EMB_DIM,), jnp.int32),
            pltpu.VMEM((EPC,), jnp.float32),
            pltpu.VMEM((EPC,), jnp.float32),
            pltpu.VMEM((EPC,), jnp.float32),
            pltpu.VMEM((EPC,), jnp.float32),
            pltpu.VMEM((EMB_DIM, LANES), jnp.float32),
            pltpu.VMEM((EMB_DIM, LANES), jnp.float32),
            pltpu.VMEM((e_per_w,), jnp.float32),
            pltpu.SemaphoreType.DMA,
            pltpu.SemaphoreType.DMA,
        ],
    )
    def score_kernel(sid_hbm, did_hbm, et_hbm, flat_hbm, tp_hbm, dp_hbm,
                     out_hbm, sid_v, did_v, et_v, eis_v, eid_v,
                     sf_a, sf_b, df_a, df_b, tp_v, dp_v, scores_v,
                     sem_a, sem_b):
        wid = lax.axis_index("s") * nc + lax.axis_index("c")
        ebase = wid * e_per_w

        pltpu.sync_copy(sid_hbm.at[pl.ds(ebase, e_per_w)], sid_v)
        pltpu.sync_copy(did_hbm.at[pl.ds(ebase, e_per_w)], did_v)
        pltpu.sync_copy(et_hbm.at[pl.ds(ebase, e_per_w)], et_v)
        pltpu.sync_copy(tp_hbm, tp_v)
        pltpu.sync_copy(dp_hbm, dp_v)

        cconst = [(c // 8) * GSTRIDE + (c % 8) * 128 for c in range(EMB_DIM)]

        def build_body(b, carry):
            chunk = b // blocks_per_chunk
            off0 = chunk * EPC + (b - chunk * blocks_per_chunk) * LANES
            for ids, ei in ((sid_v, eis_v), (did_v, eid_v)):
                ivec = ids[pl.ds(b * LANES, LANES)]
                base = (lax.shift_right_logical(ivec, 7) * 1024
                        + (ivec & 127))
                for c in range(EMB_DIM):
                    ei[pl.ds(off0 + c * CEDGES, LANES)] = base + cconst[c]
            return carry
        lax.fori_loop(0, e_per_w // LANES, build_body, 0)

        def fire_chunk(chunk, ei, dstbuf, sem):
            pltpu.async_copy(
                flat_hbm.at[ei.at[pl.ds(chunk * EPC, EPC)]],
                dstbuf,
                sem)

        def wait_chunk(chunk, ei, dstbuf, sem):
            pltpu.make_async_copy(
                flat_hbm.at[ei.at[pl.ds(chunk * EPC, EPC)]],
                dstbuf,
                sem).wait()

        def compute_chunk(chunk, sbuf, dbuf):
            def cb(b, carry):
                e0 = chunk * CEDGES + b * LANES
                et_vec = et_v[pl.ds(e0, LANES)]
                acc = jnp.zeros((LANES,), jnp.float32)
                for c in range(EMB_DIM):
                    sl = pl.ds(c * CEDGES + b * LANES, LANES)
                    sv = sbuf[sl]
                    dv = dbuf[sl]
                    tt = _shuffle(tp_v[c], et_vec)
                    dd = _shuffle(dp_v[c], et_vec)
                    acc = acc + (sv + tt) * (dv * dd)
                scores_v[pl.ds(e0, LANES)] = acc
                return carry
            lax.fori_loop(0, blocks_per_chunk, cb, 0)

        sbufs = (sf_a, sf_b)
        dbufs = (df_a, df_b)
        fire_chunk(0, eis_v, sbufs[0], sem_a)
        fire_chunk(0, eid_v, dbufs[0], sem_b)
        for chunk in range(n_chunks):
            if chunk + 1 < n_chunks:
                fire_chunk(chunk + 1, eis_v, sbufs[(chunk + 1) % 2], sem_a)
                fire_chunk(chunk + 1, eid_v, dbufs[(chunk + 1) % 2], sem_b)
            wait_chunk(chunk, eis_v, sbufs[chunk % 2], sem_a)
            wait_chunk(chunk, eid_v, dbufs[chunk % 2], sem_b)
            compute_chunk(chunk, sbufs[chunk % 2], dbufs[chunk % 2])

        pltpu.sync_copy(scores_v, out_hbm.at[pl.ds(ebase, e_per_w)])

    return score_kernel


def kernel(src_dst_pairs, condensed_edge_types, table, src_translation, dst_diag):
    batch = condensed_edge_types.shape[0]
    pairs2 = jnp.asarray(src_dst_pairs, jnp.int32).reshape(batch, 2)
    src_ids = pairs2[:, 0]
    dst_ids = pairs2[:, 1]
    tpad = jnp.pad(src_translation.T, ((0, 0), (0, LANES - NUM_EDGE_TYPES)))
    dpad = jnp.pad(dst_diag.T, ((0, 0), (0, LANES - NUM_EDGE_TYPES)))
    raw = _build_pack()(table.T)
    flat = raw.reshape(-1)
    return _build_score(batch)(
        src_ids,
        dst_ids,
        jnp.asarray(condensed_edge_types, jnp.int32),
        flat,
        tpad,
        dpad,
    )
